# Initial kernel scaffold; baseline (speedup 1.0000x reference)
import jax
import jax.numpy as jnp
from jax.experimental import pallas as pl


def kernel(poses):
    raise NotImplementedError("write your pallas kernel here")


# speedup vs baseline: 61.8018x; 61.8018x over previous
import jax
import jax.numpy as jnp
from jax.experimental import pallas as pl

_ROWS = 12800
_IN_LANES = 512
_OUT_LANES = 256
_BLOCK_ROWS = 800


def _body(in_ref, out_ref):
    v = in_ref[...]
    e = jnp.floor(v * 16.0)
    lane = jax.lax.broadcasted_iota(jnp.int32, e.shape, 1)
    even = (lane % 2) == 0
    lo = jnp.where(even, 0.0, -16.0)
    hi = jnp.where(even, 127.0, 15.0)
    e = jnp.clip(e, lo, hi)
    r = jax.lax.broadcasted_iota(jnp.int32, (_IN_LANES, _OUT_LANES), 0)
    c = jax.lax.broadcasted_iota(jnp.int32, (_IN_LANES, _OUT_LANES), 1)
    w = jnp.where(r == 2 * c, 32.0, 0.0) + jnp.where(r == 2 * c + 1, 1.0, 0.0)
    acc = jax.lax.dot_general(
        e.astype(jnp.bfloat16), w.astype(jnp.bfloat16),
        (((1,), (0,)), ((), ())), preferred_element_type=jnp.float32)
    out_ref[...] = acc.astype(jnp.int32) + 16


def kernel(poses):
    flat = jnp.reshape(poses, (_ROWS, _IN_LANES))
    out = pl.pallas_call(
        _body,
        grid=(_ROWS // _BLOCK_ROWS,),
        in_specs=[pl.BlockSpec((_BLOCK_ROWS, _IN_LANES), lambda i: (i, 0))],
        out_specs=pl.BlockSpec((_BLOCK_ROWS, _OUT_LANES), lambda i: (i, 0)),
        out_shape=jax.ShapeDtypeStruct((_ROWS, _OUT_LANES), jnp.int32),
    )(flat)
    return jnp.reshape(out, (16384, 200, 1))
